# baseline (speedup 1.0000x reference)
import jax
import jax.numpy as jnp
from jax import lax
from jax.experimental import pallas as pl
from jax.experimental.pallas import tpu as pltpu
import jax.experimental.pallas.tpu_sc as plsc

N = 10000
E = 320000
F = 128

NC = 2
NS = 16
NW = NC * NS

CH = 96
EPW = E // NW
NCHUNK = EPW // CH
TAIL = EPW - NCHUNK * CH
NB = 4
RPT = 624
REM = N - NS * RPT

_DNUMS = lax.GatherDimensionNumbers(
    offset_dims=(), collapsed_slice_dims=(0,), start_index_map=(0,))


def _bcast_lane(vec, e):
  idx = jnp.full((16, 1), e, jnp.int32)
  return lax.gather(vec, idx, _DNUMS, (1,),
                    mode=lax.GatherScatterMode.PROMISE_IN_BOUNDS)


def _expand_scale_rows(dst_v, src_v, wrow, nedge):
  def group_body(g, carry):
    wv = wrow[pl.ds(g * 16, 16)]
    for e in range(16):
      wb = _bcast_lane(wv, e)
      row = g * 16 + e
      for b in range(F // 32):
        v = src_v[row, pl.ds(b * 16, 16)]
        sh = jnp.full((16,), 65536, jnp.int32)
        msk = jnp.full((16,), -65536, jnp.int32)
        lo = lax.bitcast_convert_type(v * sh, jnp.float32)
        hi = lax.bitcast_convert_type(jnp.bitwise_and(v, msk), jnp.float32)
        dst_v[row, pl.ds(b * 32, 16)] = lo * wb
        dst_v[row, pl.ds(b * 32 + 16, 16)] = hi * wb
    return carry

  lax.fori_loop(0, nedge // 16, group_body, 0)


def _spmm_body(x_hbm, ei_hbm, w_hbm, out_hbm,
               srcv, dstv, wv, rows, fbuf, msrc, mdst, mw, acc,
               isem0, isem1, isem2, isem3,
               gsem0, gsem1, gsem2, gsem3,
               ssem0, ssem1, ssem2, ssem3):
  c = lax.axis_index("c")
  s = lax.axis_index("s")
  wid = c * NS + s
  ebase = wid * EPW
  rbase = s * RPT
  isems = (isem0, isem1, isem2, isem3)
  gsems = (gsem0, gsem1, gsem2, gsem3)
  ssems = (ssem0, ssem1, ssem2, ssem3)

  def idx_load(i, q):
    off = ebase + i * CH
    pltpu.async_copy(ei_hbm.at[pl.ds(off, CH)], srcv.at[q], isems[q])
    pltpu.async_copy(ei_hbm.at[pl.ds(E + off, CH)], dstv.at[q], isems[q])
    pltpu.async_copy(w_hbm.at[pl.ds(off, CH)], wv.at[q], isems[q])

  def idx_wait(i, q):
    off = ebase + i * CH
    pltpu.make_async_copy(ei_hbm.at[pl.ds(off, CH)], srcv.at[q],
                          isems[q]).wait()
    pltpu.make_async_copy(ei_hbm.at[pl.ds(E + off, CH)], dstv.at[q],
                          isems[q]).wait()
    pltpu.make_async_copy(w_hbm.at[pl.ds(off, CH)], wv.at[q],
                          isems[q]).wait()

  def gather_start(q):
    pltpu.async_copy(x_hbm.at[srcv.at[q]], rows.at[q], gsems[q])

  def gather_wait(q):
    pltpu.make_async_copy(x_hbm.at[srcv.at[q]], rows.at[q], gsems[q]).wait()

  def scatter_start(q, f):
    pltpu.async_copy(fbuf.at[f], acc.at[dstv.at[q]], ssems[q], add=True)

  def scatter_wait(q, f):
    pltpu.make_async_copy(fbuf.at[f], acc.at[dstv.at[q]], ssems[q]).wait()

  idx_load(0, 0)
  idx_load(1, 1)
  idx_load(2, 2)

  def zfill(rr, carry):
    zero = jnp.zeros((16,), jnp.float32)
    for j in range(F // 16):
      fbuf[0, rr, pl.ds(j * 16, 16)] = zero
    return carry

  lax.fori_loop(0, CH, zfill, 0)
  for k in range(RPT // CH):
    pltpu.sync_copy(fbuf.at[0], acc.at[pl.ds(rbase + k * CH, CH)])
  lastoff = (RPT // CH) * CH
  pltpu.sync_copy(fbuf.at[0, pl.ds(0, RPT - lastoff)],
                  acc.at[pl.ds(rbase + lastoff, RPT - lastoff)])

  @pl.when(s == NS - 1)
  def _():
    pltpu.sync_copy(fbuf.at[0, pl.ds(0, REM)],
                    acc.at[pl.ds(NS * RPT, REM)])

  idx_wait(0, 0)
  gather_start(0)
  idx_wait(1, 1)
  gather_start(1)

  plsc.subcore_barrier()

  def chunk_step(i, k, f, wait_prev, load_next, gather_next):
    gather_wait(k)
    if wait_prev:
      scatter_wait((k + 3) % NB, 1 - f)
    _expand_scale_rows(fbuf.at[f], rows.at[k], wv.at[k], CH)
    scatter_start(k, f)
    if load_next:
      idx_load(i + 3, (k + 3) % NB)
    if gather_next:
      q2 = (k + 2) % NB
      idx_wait(i + 2, q2)
      gather_start(q2)

  chunk_step(0, 0, 0, False, True, True)
  chunk_step(1, 1, 1, False, True, True)
  chunk_step(2, 2, 0, True, True, True)
  chunk_step(3, 3, 1, True, True, True)

  def loop_body(g, carry):
    i0 = 4 + 4 * g
    for k in range(NB):
      chunk_step(i0 + k, k, k % 2, True, True, True)
    return carry

  lax.fori_loop(0, (NCHUNK - 8) // NB, loop_body, 0)

  i0 = NCHUNK - 4
  chunk_step(i0, 0, 0, True, True, True)
  chunk_step(i0 + 1, 1, 1, True, False, True)
  chunk_step(i0 + 2, 2, 0, True, False, False)
  chunk_step(i0 + 3, 3, 1, True, False, False)
  scatter_wait(3, 1)

  toff = ebase + NCHUNK * CH
  pltpu.sync_copy(ei_hbm.at[pl.ds(toff, TAIL)], msrc)
  pltpu.sync_copy(ei_hbm.at[pl.ds(E + toff, TAIL)], mdst)
  pltpu.sync_copy(w_hbm.at[pl.ds(toff, TAIL)], mw)
  mrows = rows.at[0, pl.ds(0, TAIL)]
  pltpu.async_copy(x_hbm.at[msrc], mrows, gsem0).wait()
  wv16 = mw[...]
  for e in range(TAIL):
    wb = _bcast_lane(wv16, e)
    for b in range(F // 32):
      v = rows[0, e, pl.ds(b * 16, 16)]
      sh = jnp.full((16,), 65536, jnp.int32)
      msk = jnp.full((16,), -65536, jnp.int32)
      lo = lax.bitcast_convert_type(v * sh, jnp.float32)
      hi = lax.bitcast_convert_type(jnp.bitwise_and(v, msk), jnp.float32)
      fbuf[0, e, pl.ds(b * 32, 16)] = lo * wb
      fbuf[0, e, pl.ds(b * 32 + 16, 16)] = hi * wb
  pltpu.sync_copy(fbuf.at[0, pl.ds(0, TAIL)], acc.at[mdst], add=True)

  plsc.subcore_barrier()

  pltpu.sync_copy(acc.at[pl.ds(rbase, RPT)], out_hbm.at[c, pl.ds(rbase, RPT)])

  @pl.when(s == NS - 1)
  def _():
    pltpu.sync_copy(acc.at[pl.ds(NS * RPT, REM)],
                    out_hbm.at[c, pl.ds(NS * RPT, REM)])


@jax.jit
def _spmm_sc(x, edge_index, edge_weight):
  mesh = plsc.VectorSubcoreMesh(core_axis_name="c", subcore_axis_name="s")
  return pl.kernel(
      _spmm_body,
      out_type=jax.ShapeDtypeStruct((NC, N, F), jnp.float32),
      mesh=mesh,
      compiler_params=pltpu.CompilerParams(use_tc_tiling_on_sc=False),
      scratch_types=[
          pltpu.VMEM((NB, CH), jnp.int32),
          pltpu.VMEM((NB, CH), jnp.int32),
          pltpu.VMEM((NB, CH), jnp.float32),
          pltpu.VMEM((NB, CH, F // 2), jnp.int32),
          pltpu.VMEM((2, CH, F), jnp.float32),
          pltpu.VMEM((TAIL,), jnp.int32),
          pltpu.VMEM((TAIL,), jnp.int32),
          pltpu.VMEM((TAIL,), jnp.float32),
          pltpu.VMEM_SHARED((N, F), jnp.float32),
          pltpu.SemaphoreType.DMA,
          pltpu.SemaphoreType.DMA,
          pltpu.SemaphoreType.DMA,
          pltpu.SemaphoreType.DMA,
          pltpu.SemaphoreType.DMA,
          pltpu.SemaphoreType.DMA,
          pltpu.SemaphoreType.DMA,
          pltpu.SemaphoreType.DMA,
          pltpu.SemaphoreType.DMA,
          pltpu.SemaphoreType.DMA,
          pltpu.SemaphoreType.DMA,
          pltpu.SemaphoreType.DMA,
      ],
  )(x, edge_index, edge_weight)


def _swz_body(x_ref, o_ref):
  x = x_ref[...]
  for b in range(F // 32):
    a = lax.convert_element_type(x[:, 32 * b:32 * b + 16], jnp.bfloat16)
    cc = lax.convert_element_type(x[:, 32 * b + 16:32 * b + 32], jnp.bfloat16)
    ai = lax.convert_element_type(
        lax.bitcast_convert_type(a, jnp.uint16), jnp.uint32)
    ci = lax.convert_element_type(
        lax.bitcast_convert_type(cc, jnp.uint16), jnp.uint32)
    w = jnp.bitwise_or(ai, lax.shift_left(ci, jnp.uint32(16)))
    o_ref[:, 16 * b:16 * b + 16] = lax.bitcast_convert_type(w, jnp.int32)


@jax.jit
def _swizzle_tc(x):
  blk = 2000
  return pl.pallas_call(
      _swz_body,
      grid=(N // blk,),
      in_specs=[pl.BlockSpec((blk, F), lambda i: (i, 0))],
      out_specs=pl.BlockSpec((blk, F // 2), lambda i: (i, 0)),
      out_shape=jax.ShapeDtypeStruct((N, F // 2), jnp.int32),
  )(x)


def _dense_body(x_ref, nb0_ref, nb1_ref, norm_ref, head_ref,
                g1_ref, g2_ref, b1_ref, b2_ref, r_ref, w_ref,
                hk_ref, m_ref):
  x = x_ref[...]
  nb = nb0_ref[...] + nb1_ref[...]
  ga = jnp.dot(x, g1_ref[...], preferred_element_type=jnp.float32)
  ga += jnp.dot(nb, g2_ref[...], preferred_element_type=jnp.float32)
  gamma = jnp.where(ga >= 0, ga, 0.2 * ga) + 1.0
  be = jnp.dot(x, b1_ref[...], preferred_element_type=jnp.float32)
  be += jnp.dot(nb, b2_ref[...], preferred_element_type=jnp.float32)
  beta = jnp.where(be >= 0, be, 0.2 * be)
  m = x + gamma * r_ref[...] + beta - nb
  m_ref[...] = m
  sel = jnp.where(head_ref[0, 0] != 0, 0.0, 1.0)
  t = x + nb + sel * m
  hk = jnp.dot(t, w_ref[...], preferred_element_type=jnp.float32)
  hk_ref[...] = hk / (norm_ref[...] + 1.0)


BLK = 1000


@jax.jit
def _dense_tc(x, nb0, nb1, norm, head, G1, G2, B1, B2, r, W):
  grid = (N // BLK,)
  row_spec = pl.BlockSpec((BLK, F), lambda i: (i, 0))
  full_spec = pl.BlockSpec((F, F), lambda i: (0, 0))
  return pl.pallas_call(
      _dense_body,
      grid=grid,
      in_specs=[
          row_spec, row_spec, row_spec,
          pl.BlockSpec((BLK, 1), lambda i: (i, 0)),
          pl.BlockSpec(memory_space=pltpu.SMEM),
          full_spec, full_spec, full_spec, full_spec,
          pl.BlockSpec((1, F), lambda i: (0, 0)),
          full_spec,
      ],
      out_specs=[row_spec, row_spec],
      out_shape=[
          jax.ShapeDtypeStruct((N, F), jnp.float32),
          jax.ShapeDtypeStruct((N, F), jnp.float32),
      ],
  )(x, nb0, nb1, norm, head, G1, G2, B1, B2, r, W)


def kernel(x, edge_index, edge_weight, norm, head, G1, G2, B1, B2, r, W):
  x_swz = _swizzle_tc(x)
  partial = _spmm_sc(x_swz, jnp.reshape(edge_index, (2 * E,)), edge_weight)
  head_arr = jnp.reshape(jnp.asarray(head, jnp.int32), (1, 1))
  hk, m = _dense_tc(x, partial[0], partial[1], norm, head_arr,
                    G1, G2, B1, B2, r, W)
  return (hk, m)


# speedup vs baseline: 2.1271x; 2.1271x over previous
import jax
import jax.numpy as jnp
from jax import lax
from jax.experimental import pallas as pl
from jax.experimental.pallas import tpu as pltpu
import jax.experimental.pallas.tpu_sc as plsc

N = 10000
E = 320000
F = 128

NC = 2
NS = 16
NW = NC * NS

CH = 96
EPW = E // NW
NCHUNK = EPW // CH
TAIL = EPW - NCHUNK * CH
NB = 4
RPT = 624
REM = N - NS * RPT

_DNUMS = lax.GatherDimensionNumbers(
    offset_dims=(), collapsed_slice_dims=(0,), start_index_map=(0,))


def _bcast_lane(vec, e):
  idx = jnp.full((16, 1), e, jnp.int32)
  return lax.gather(vec, idx, _DNUMS, (1,),
                    mode=lax.GatherScatterMode.PROMISE_IN_BOUNDS)


def _scale_rows(rows_v, wrow, nedge):
  def group_body(g, carry):
    wv = wrow[pl.ds(g * 16, 16)]
    for e in range(16):
      wb = _bcast_lane(wv, e)
      row = g * 16 + e
      for j in range(F // 16):
        sl = pl.ds(j * 16, 16)
        rows_v[row, sl] = rows_v[row, sl] * wb
    return carry

  lax.fori_loop(0, nedge // 16, group_body, 0)


def _spmm_body(x_hbm, ei_hbm, w_hbm, out_hbm,
               srcv, dstv, wv, rows, msrc, mdst, mw, acc,
               isem0, isem1, isem2, isem3,
               gsem0, gsem1, gsem2, gsem3,
               ssem0, ssem1, ssem2, ssem3):
  c = lax.axis_index("c")
  s = lax.axis_index("s")
  wid = c * NS + s
  ebase = wid * EPW
  rbase = s * RPT
  isems = (isem0, isem1, isem2, isem3)
  gsems = (gsem0, gsem1, gsem2, gsem3)
  ssems = (ssem0, ssem1, ssem2, ssem3)

  def idx_load(i, q):
    off = ebase + i * CH
    pltpu.async_copy(ei_hbm.at[pl.ds(off, CH)], srcv.at[q], isems[q])
    pltpu.async_copy(ei_hbm.at[pl.ds(E + off, CH)], dstv.at[q], isems[q])
    pltpu.async_copy(w_hbm.at[pl.ds(off, CH)], wv.at[q], isems[q])

  def idx_wait(i, q):
    off = ebase + i * CH
    pltpu.make_async_copy(ei_hbm.at[pl.ds(off, CH)], srcv.at[q],
                          isems[q]).wait()
    pltpu.make_async_copy(ei_hbm.at[pl.ds(E + off, CH)], dstv.at[q],
                          isems[q]).wait()
    pltpu.make_async_copy(w_hbm.at[pl.ds(off, CH)], wv.at[q],
                          isems[q]).wait()

  def gather_start(q):
    pltpu.async_copy(x_hbm.at[srcv.at[q]], rows.at[q], gsems[q])

  def gather_wait(q):
    pltpu.make_async_copy(x_hbm.at[srcv.at[q]], rows.at[q], gsems[q]).wait()

  def scatter_start(q):
    pltpu.async_copy(rows.at[q], acc.at[dstv.at[q]], ssems[q], add=True)

  def scatter_wait(q):
    pltpu.make_async_copy(rows.at[q], acc.at[dstv.at[q]], ssems[q]).wait()

  idx_load(0, 0)
  idx_load(1, 1)
  idx_load(2, 2)

  def zfill(rr, carry):
    zero = jnp.zeros((16,), jnp.float32)
    for j in range(F // 16):
      rows[3, rr, pl.ds(j * 16, 16)] = zero
    return carry

  lax.fori_loop(0, CH, zfill, 0)
  for k in range(RPT // CH):
    pltpu.sync_copy(rows.at[3], acc.at[pl.ds(rbase + k * CH, CH)])
  lastoff = (RPT // CH) * CH
  pltpu.sync_copy(rows.at[3, pl.ds(0, RPT - lastoff)],
                  acc.at[pl.ds(rbase + lastoff, RPT - lastoff)])

  @pl.when(s == NS - 1)
  def _():
    pltpu.sync_copy(rows.at[3, pl.ds(0, REM)],
                    acc.at[pl.ds(NS * RPT, REM)])

  idx_wait(0, 0)
  gather_start(0)
  idx_wait(1, 1)
  gather_start(1)

  plsc.subcore_barrier()

  def chunk_step(i, k, wait_prev, load_next, gather_next):
    gather_wait(k)
    _scale_rows(rows.at[k], wv.at[k], CH)
    if wait_prev:
      scatter_wait((k + 3) % NB)
    scatter_start(k)
    if load_next:
      idx_load(i + 3, (k + 3) % NB)
    if gather_next:
      q2 = (k + 2) % NB
      idx_wait(i + 2, q2)
      gather_start(q2)

  chunk_step(0, 0, False, True, True)
  chunk_step(1, 1, True, True, True)
  chunk_step(2, 2, True, True, True)
  chunk_step(3, 3, True, True, True)

  def loop_body(g, carry):
    i0 = 4 + 4 * g
    for k in range(NB):
      chunk_step(i0 + k, k, True, True, True)
    return carry

  lax.fori_loop(0, (NCHUNK - 8) // NB, loop_body, 0)

  i0 = NCHUNK - 4
  chunk_step(i0, 0, True, True, True)
  chunk_step(i0 + 1, 1, True, False, True)
  chunk_step(i0 + 2, 2, True, False, False)
  chunk_step(i0 + 3, 3, True, False, False)
  scatter_wait(3)

  toff = ebase + NCHUNK * CH
  pltpu.sync_copy(ei_hbm.at[pl.ds(toff, TAIL)], msrc)
  pltpu.sync_copy(ei_hbm.at[pl.ds(E + toff, TAIL)], mdst)
  pltpu.sync_copy(w_hbm.at[pl.ds(toff, TAIL)], mw)
  mrows = rows.at[0, pl.ds(0, TAIL)]
  pltpu.async_copy(x_hbm.at[msrc], mrows, gsem0).wait()
  wv16 = mw[...]
  for e in range(TAIL):
    wb = _bcast_lane(wv16, e)
    for j in range(F // 16):
      sl = pl.ds(j * 16, 16)
      rows[0, e, sl] = rows[0, e, sl] * wb
  pltpu.sync_copy(mrows, acc.at[mdst], add=True)

  plsc.subcore_barrier()

  pltpu.sync_copy(acc.at[pl.ds(rbase, RPT)], out_hbm.at[c, pl.ds(rbase, RPT)])

  @pl.when(s == NS - 1)
  def _():
    pltpu.sync_copy(acc.at[pl.ds(NS * RPT, REM)],
                    out_hbm.at[c, pl.ds(NS * RPT, REM)])


@jax.jit
def _spmm_sc(x, edge_index, edge_weight):
  mesh = plsc.VectorSubcoreMesh(core_axis_name="c", subcore_axis_name="s")
  return pl.kernel(
      _spmm_body,
      out_type=jax.ShapeDtypeStruct((NC, N, F), jnp.float32),
      mesh=mesh,
      scratch_types=[
          pltpu.VMEM((NB, CH), jnp.int32),
          pltpu.VMEM((NB, CH), jnp.int32),
          pltpu.VMEM((NB, CH), jnp.float32),
          pltpu.VMEM((NB, CH, F), jnp.float32),
          pltpu.VMEM((TAIL,), jnp.int32),
          pltpu.VMEM((TAIL,), jnp.int32),
          pltpu.VMEM((TAIL,), jnp.float32),
          pltpu.VMEM_SHARED((N, F), jnp.float32),
          pltpu.SemaphoreType.DMA,
          pltpu.SemaphoreType.DMA,
          pltpu.SemaphoreType.DMA,
          pltpu.SemaphoreType.DMA,
          pltpu.SemaphoreType.DMA,
          pltpu.SemaphoreType.DMA,
          pltpu.SemaphoreType.DMA,
          pltpu.SemaphoreType.DMA,
          pltpu.SemaphoreType.DMA,
          pltpu.SemaphoreType.DMA,
          pltpu.SemaphoreType.DMA,
          pltpu.SemaphoreType.DMA,
      ],
  )(x, edge_index, edge_weight)


def _dense_body(x_ref, nb0_ref, nb1_ref, norm_ref, head_ref,
                g1_ref, g2_ref, b1_ref, b2_ref, r_ref, w_ref,
                hk_ref, m_ref):
  x = x_ref[...]
  nb = nb0_ref[...] + nb1_ref[...]
  ga = jnp.dot(x, g1_ref[...], preferred_element_type=jnp.float32)
  ga += jnp.dot(nb, g2_ref[...], preferred_element_type=jnp.float32)
  gamma = jnp.where(ga >= 0, ga, 0.2 * ga) + 1.0
  be = jnp.dot(x, b1_ref[...], preferred_element_type=jnp.float32)
  be += jnp.dot(nb, b2_ref[...], preferred_element_type=jnp.float32)
  beta = jnp.where(be >= 0, be, 0.2 * be)
  m = x + gamma * r_ref[...] + beta - nb
  m_ref[...] = m
  sel = jnp.where(head_ref[0, 0] != 0, 0.0, 1.0)
  t = x + nb + sel * m
  hk = jnp.dot(t, w_ref[...], preferred_element_type=jnp.float32)
  hk_ref[...] = hk / (norm_ref[...] + 1.0)


BLK = 1000


@jax.jit
def _dense_tc(x, nb0, nb1, norm, head, G1, G2, B1, B2, r, W):
  grid = (N // BLK,)
  row_spec = pl.BlockSpec((BLK, F), lambda i: (i, 0))
  full_spec = pl.BlockSpec((F, F), lambda i: (0, 0))
  return pl.pallas_call(
      _dense_body,
      grid=grid,
      in_specs=[
          row_spec, row_spec, row_spec,
          pl.BlockSpec((BLK, 1), lambda i: (i, 0)),
          pl.BlockSpec(memory_space=pltpu.SMEM),
          full_spec, full_spec, full_spec, full_spec,
          pl.BlockSpec((1, F), lambda i: (0, 0)),
          full_spec,
      ],
      out_specs=[row_spec, row_spec],
      out_shape=[
          jax.ShapeDtypeStruct((N, F), jnp.float32),
          jax.ShapeDtypeStruct((N, F), jnp.float32),
      ],
  )(x, nb0, nb1, norm, head, G1, G2, B1, B2, r, W)


def kernel(x, edge_index, edge_weight, norm, head, G1, G2, B1, B2, r, W):
  partial = _spmm_sc(x, jnp.reshape(edge_index, (2 * E,)), edge_weight)
  head_arr = jnp.reshape(jnp.asarray(head, jnp.int32), (1, 1))
  hk, m = _dense_tc(x, partial[0], partial[1], norm, head_arr,
                    G1, G2, B1, B2, r, W)
  return (hk, m)
